# Initial kernel scaffold; baseline (speedup 1.0000x reference)
import functools

import jax
import jax.numpy as jnp
from jax import lax
from jax.experimental import pallas as pl
from jax.experimental.pallas import tpu as pltpu
from jax.experimental.pallas import tpu_sc as plsc

K = 16
EPS = 1e-6
SCALE_FACTOR = 640.0
B, N = 2, 2048
M = B * N

NC, NS, L = 2, 16, 16
NW = NC * NS
PTS = M // NW
T = 4
S = PTS // T
BN = 256


def _knn_body(x_ref, xt_ref, idx_ref):
    b = pl.program_id(0)
    i = pl.program_id(1)
    fx = x_ref[0, 0:1, :]
    fy = x_ref[0, 1:2, :]
    fz = x_ref[0, 2:3, :]
    sq = fx * fx + fy * fy + fz * fz
    fr = xt_ref[0, pl.ds(i * BN, BN), :]
    rx = fr[:, 0:1]
    ry = fr[:, 1:2]
    rz = fr[:, 2:3]
    sqr = rx * rx + ry * ry + rz * rz
    g = rx * fx + ry * fy + rz * fz
    d2 = (sqr - 2.0 * g) + sq
    col = lax.broadcasted_iota(jnp.int32, (BN, N), 1)
    cols = []
    for _ in range(K):
        m = jnp.min(d2, axis=1, keepdims=True)
        am = jnp.min(jnp.where(d2 == m, col, N), axis=1, keepdims=True)
        cols.append(am)
        d2 = jnp.where(col == am, jnp.inf, d2)
    idx_ref[...] = jnp.concatenate(cols, axis=1) + b * N


def _knn_call(x, xt):
    return pl.pallas_call(
        _knn_body,
        grid=(B, N // BN),
        in_specs=[
            pl.BlockSpec((1, 3, N), lambda b, i: (b, 0, 0)),
            pl.BlockSpec((1, N, 3), lambda b, i: (b, 0, 0)),
        ],
        out_specs=pl.BlockSpec((BN, K), lambda b, i: (b * (N // BN) + i, 0)),
        out_shape=jax.ShapeDtypeStruct((M, K), jnp.int32),
    )(x, xt)


def _mm_body(x_ref, w1_ref, wq_ref, wd_ref, pp_ref, qq_ref):
    x = x_ref[...]
    if x.shape[1] == 1:
        p = x * w1_ref[...]
        q = x * wq_ref[...]
    else:
        p = jnp.dot(x, w1_ref[...], preferred_element_type=jnp.float32)
        q = jnp.dot(x, wq_ref[...], preferred_element_type=jnp.float32)
    pd = jnp.dot(p, wd_ref[...], preferred_element_type=jnp.float32)
    qd = jnp.dot(q, wd_ref[...], preferred_element_type=jnp.float32)
    o = pd.shape[1]
    pp_ref[:, :o] = p
    pp_ref[:, o:] = pd
    qq_ref[:, :o] = q
    qq_ref[:, o:] = qd


def _mm_call(xr, w1t, wqt, wdt, O):
    C = xr.shape[1]
    RB = 1536
    nb = (M * 3) // RB
    return pl.pallas_call(
        _mm_body,
        grid=(nb,),
        in_specs=[
            pl.BlockSpec((RB, C), lambda i: (i, 0)),
            pl.BlockSpec((C, O), lambda i: (0, 0)),
            pl.BlockSpec((C, O), lambda i: (0, 0)),
            pl.BlockSpec((O, O), lambda i: (0, 0)),
        ],
        out_specs=[
            pl.BlockSpec((RB, 2 * O), lambda i: (i, 0)),
            pl.BlockSpec((RB, 2 * O), lambda i: (i, 0)),
        ],
        out_shape=[
            jax.ShapeDtypeStruct((M * 3, 2 * O), jnp.float32),
            jax.ShapeDtypeStruct((M * 3, 2 * O), jnp.float32),
        ],
    )(xr, w1t, wqt, wdt)


def _sc_rsqrt(s):
    i = lax.bitcast_convert_type(s, jnp.int32)
    i = jnp.int32(0x5F3759DF) - lax.shift_right_arithmetic(i, 1)
    r = lax.bitcast_convert_type(i, jnp.float32)
    r = r * (1.5 - 0.5 * s * r * r)
    r = r * (1.5 - 0.5 * s * r * r)
    return r


def _sc_layer_call(pp, qq, idxr, O):
    G = O // L
    mesh = plsc.VectorSubcoreMesh(core_axis_name="c", subcore_axis_name="s")

    def body(pp_hbm, qq_hbm, idx_hbm, out_hbm,
             idx_v, rows0, rows1, qqv0, qqv1, out_v, sg0, sg1, sq0, sq1):
        wid = lax.axis_index("s") * NC + lax.axis_index("c")
        base_pt = wid * PTS
        rows = (rows0, rows1)
        qqv = (qqv0, qqv1)
        sg = (sg0, sg1)
        sq = (sq0, sq1)

        pltpu.sync_copy(idx_hbm.at[wid], idx_v)

        def issue(s, j):
            pltpu.async_copy(pp_hbm.at[idx_v.at[s]], rows[j], sg[j])
            pltpu.async_copy(qq_hbm.at[pl.ds(base_pt + s * T, T)], qqv[j], sq[j])

        def wait(s, j):
            pltpu.make_async_copy(pp_hbm.at[idx_v.at[s]], rows[j], sg[j]).wait()
            pltpu.make_async_copy(
                qq_hbm.at[pl.ds(base_pt + s * T, T)], qqv[j], sq[j]).wait()

        def compute(rows_ref, qq_ref):
            for p in range(T):
                def g_body(g, _):
                    qx = qq_ref[p, 0, pl.ds(g * L, L)]
                    qy = qq_ref[p, 1, pl.ds(g * L, L)]
                    qz = qq_ref[p, 2, pl.ds(g * L, L)]
                    qdx = qq_ref[p, 0, pl.ds(O + g * L, L)]
                    qdy = qq_ref[p, 1, pl.ds(O + g * L, L)]
                    qdz = qq_ref[p, 2, pl.ds(O + g * L, L)]

                    def k_body(k, acc):
                        ax, ay, az = acc
                        r = p * K + k
                        hx = rows_ref[r, 0, pl.ds(g * L, L)] + qx
                        hy = rows_ref[r, 1, pl.ds(g * L, L)] + qy
                        hz = rows_ref[r, 2, pl.ds(g * L, L)] + qz
                        dx = rows_ref[r, 0, pl.ds(O + g * L, L)] + qdx
                        dy = rows_ref[r, 1, pl.ds(O + g * L, L)] + qdy
                        dz = rows_ref[r, 2, pl.ds(O + g * L, L)] + qdz
                        s2 = jnp.maximum(dx * dx + dy * dy + dz * dz, 1e-30)
                        rr = _sc_rsqrt(s2)
                        inv = 1.0 / (s2 * rr + EPS)
                        dt = (hx * dx + hy * dy + hz * dz) * inv
                        t = -0.8 * jnp.minimum(dt, 0.0) * inv
                        return (ax + hx + t * dx,
                                ay + hy + t * dy,
                                az + hz + t * dz)

                    z = jnp.zeros((L,), jnp.float32)
                    ax, ay, az = lax.fori_loop(0, K, k_body, (z, z, z))
                    out_v[p, 0, pl.ds(g * L, L)] = ax * (1.0 / K)
                    out_v[p, 1, pl.ds(g * L, L)] = ay * (1.0 / K)
                    out_v[p, 2, pl.ds(g * L, L)] = az * (1.0 / K)
                    return 0

                lax.fori_loop(0, G, g_body, 0)

        issue(0, 0)

        def pair(i, carry):
            for j in range(2):
                s = 2 * i + j
                nxt = 1 - j

                @pl.when(s + 1 < S)
                def _():
                    issue(s + 1, nxt)

                wait(s, j)
                compute(rows[j], qqv[j])
                pltpu.sync_copy(out_v, out_hbm.at[pl.ds(base_pt + s * T, T)])
            return carry

        lax.fori_loop(0, S // 2, pair, 0)

    fn = pl.kernel(
        body,
        out_type=jax.ShapeDtypeStruct((M, 3, O), jnp.float32),
        mesh=mesh,
        scratch_types=[
            pltpu.VMEM((S, T * K), jnp.int32),
            pltpu.VMEM((T * K, 3, 2 * O), jnp.float32),
            pltpu.VMEM((T * K, 3, 2 * O), jnp.float32),
            pltpu.VMEM((T, 3, 2 * O), jnp.float32),
            pltpu.VMEM((T, 3, 2 * O), jnp.float32),
            pltpu.VMEM((T, 3, O), jnp.float32),
            pltpu.SemaphoreType.DMA,
            pltpu.SemaphoreType.DMA,
            pltpu.SemaphoreType.DMA,
            pltpu.SemaphoreType.DMA,
        ],
    )
    return fn(pp, qq, idxr)


RBC = 512


def _convc_body(x1_ref, x2_ref, x3_ref, x4_ref,
                w1_ref, w2_ref, w3_ref, w4_ref, wd_ref, out_ref):
    i = pl.program_id(1)
    h = jnp.dot(x1_ref[...], w1_ref[...], preferred_element_type=jnp.float32)
    h += jnp.dot(x2_ref[...], w2_ref[...], preferred_element_type=jnp.float32)
    h += jnp.dot(x3_ref[...], w3_ref[...], preferred_element_type=jnp.float32)
    h += jnp.dot(x4_ref[...], w4_ref[...], preferred_element_type=jnp.float32)
    dvec = jnp.dot(h, wd_ref[...], preferred_element_type=jnp.float32)
    d3 = dvec.reshape(RBC, 3)
    dx = d3[:, 0:1]
    dy = d3[:, 1:2]
    dz = d3[:, 2:3]
    nrm = jnp.sqrt(dx * dx + dy * dy + dz * dz)
    inv = 1.0 / (nrm + EPS)
    dnx = dx * inv
    dny = dy * inv
    dnz = dz * inv
    h3 = h.reshape(RBC, 3, 128)
    hx = h3[:, 0, :]
    hy = h3[:, 1, :]
    hz = h3[:, 2, :]
    dt = hx * dnx + hy * dny + hz * dnz
    t = -0.8 * jnp.minimum(dt, 0.0)

    @pl.when(i == 0)
    def _():
        out_ref[...] = jnp.zeros_like(out_ref)

    out_ref[:, 0, :] += jnp.sum(hx + t * dnx, axis=0, keepdims=True)
    out_ref[:, 1, :] += jnp.sum(hy + t * dny, axis=0, keepdims=True)
    out_ref[:, 2, :] += jnp.sum(hz + t * dnz, axis=0, keepdims=True)


def _convc_call(x1, x2, x3, x4, w1, w2, w3, w4, wd):
    nb = N // RBC

    def rspec(c):
        return pl.BlockSpec((RBC * 3, c), lambda b, i: (b * nb + i, 0))

    def wspec(c):
        return pl.BlockSpec((c, 128), lambda b, i: (0, 0))

    return pl.pallas_call(
        _convc_body,
        grid=(B, nb),
        in_specs=[rspec(32), rspec(64), rspec(128), rspec(128),
                  wspec(32), wspec(64), wspec(128), wspec(128),
                  pl.BlockSpec((128, 1), lambda b, i: (0, 0))],
        out_specs=pl.BlockSpec((1, 3, 128), lambda b, i: (b, 0, 0)),
        out_shape=jax.ShapeDtypeStruct((B, 3, 128), jnp.float32),
    )(x1, x2, x3, x4, w1, w2, w3, w4, wd)


def _vl(w, x):
    return jnp.einsum('oi,bi...->bo...', w, x)


def _vact(wdir, x):
    d = _vl(wdir, x)
    dn = d / (jnp.linalg.norm(d, axis=2, keepdims=True) + EPS)
    dot = jnp.sum(x * dn, axis=2, keepdims=True)
    acted = jnp.where(dot >= 0, dot, 0.2 * dot)
    return x + (acted - dot) * dn


def _vlna(p, prefix, x):
    return _vact(p[prefix + '_dir'], _vl(p[prefix + '_lin'], x))


def _cevn(x):
    n = jnp.linalg.norm(x, axis=2, keepdims=True)
    d = x / (n + EPS)
    nrm = n / (jnp.linalg.norm(n, axis=1, keepdims=True) + EPS)
    return d * nrm


def kernel(x, label, params):
    p = params
    xt = jnp.transpose(x, (0, 2, 1))
    idxg = _knn_call(x, xt)
    idxr = idxg.reshape(NW, S, T * K)

    X = xt.reshape(M, 3, 1)
    feats = []
    for name, C, O in (('conv1', 1, 32), ('conv2', 32, 64),
                       ('conv3', 64, 128), ('conv4', 128, 128)):
        lin = p[name + '_lin']
        w1t = jnp.transpose(lin[:, :C])
        wqt = jnp.transpose(lin[:, C:] - lin[:, :C])
        wdt = jnp.transpose(p[name + '_dir'])
        pp, qq = _mm_call(X.reshape(M * 3, C), w1t, wqt, wdt, O)
        X = _sc_layer_call(pp.reshape(M, 3, 2 * O), qq.reshape(M, 3, 2 * O),
                           idxr, O)
        feats.append(X)

    x1, x2, x3, x4 = feats
    wc = p['convc_lin']
    ygs = _convc_call(
        x1.reshape(M * 3, 32), x2.reshape(M * 3, 64),
        x3.reshape(M * 3, 128), x4.reshape(M * 3, 128),
        jnp.transpose(wc[:, :32]), jnp.transpose(wc[:, 32:96]),
        jnp.transpose(wc[:, 96:224]), jnp.transpose(wc[:, 224:]),
        jnp.transpose(p['convc_dir']))
    yg = jnp.transpose(ygs, (0, 2, 1)) * (1.0 / N)

    so3f = _cevn(yg)
    z_so3 = _vl(p['fc_O'], so3f)
    r_pred = jnp.swapaxes(z_so3, 2, 1)
    u, _, vh = jnp.linalg.svd(r_pred, full_matrices=False)
    so3 = jnp.swapaxes(jnp.einsum('bij,bjk->bik', u, vh), 2, 1)
    scale = jnp.mean(jnp.linalg.norm(yg, axis=-1) + 1e-12, axis=1) * SCALE_FACTOR
    xg = yg[..., None]
    h = _vlna(p, 'ctr0', xg)
    h = _vlna(p, 'ctr1', h)
    center = (_vl(p['ctr_out'], h) + _vl(p['ctr_sc'], xg))[..., 0] * SCALE_FACTOR
    yn = _cevn(yg)
    m = _vlna(p, 'mean2', _vlna(p, 'mean1', yn))
    lv = _vlna(p, 'logvar2', _vlna(p, 'logvar1', yn))
    so3_m = _vl(p['fc_inv_mean'], m[..., None])[..., 0]
    inv_mean = jnp.sum(m * so3_m, axis=-1)
    so3_lv = _vl(p['fc_inv_logvar'], lv[..., None])[..., 0]
    inv_logvar = jnp.sum(lv * so3_lv, axis=-1)
    return inv_mean, inv_logvar, so3, scale, center


# speedup vs baseline: 6.5023x; 6.5023x over previous
import jax
import jax.numpy as jnp
from jax import lax
from jax.experimental import pallas as pl
from jax.experimental.pallas import tpu as pltpu
from jax.experimental.pallas import tpu_sc as plsc

K = 16
EPS = 1e-6
SCALE_FACTOR = 640.0
B, N = 2, 2048
M = B * N

NC, NS = 2, 16
NW = NC * NS
ROWS_W = M * K // NW
GCH = 128
GS = ROWS_W // GCH
BN = 256
RB = 128
RBC = 512


def _r128(w):
    return (w + 127) // 128 * 128


def _knn_body(x_ref, xt_ref, idx_ref):
    b = pl.program_id(0)
    i = pl.program_id(1)
    fx = x_ref[0, 0:1, :]
    fy = x_ref[0, 1:2, :]
    fz = x_ref[0, 2:3, :]
    sq = fx * fx + fy * fy + fz * fz
    fr = xt_ref[0, pl.ds(i * BN, BN), :]
    rx = fr[:, 0:1]
    ry = fr[:, 1:2]
    rz = fr[:, 2:3]
    sqr = rx * rx + ry * ry + rz * rz
    g = lax.dot_general(fr, x_ref[0], (((1,), (0,)), ((), ())),
                        preferred_element_type=jnp.float32)
    d2 = (sqr - 2.0 * g) + sq
    col = lax.broadcasted_iota(jnp.int32, (BN, N), 1)
    cols = []
    for _ in range(K):
        m = jnp.min(d2, axis=1, keepdims=True)
        am = jnp.min(jnp.where(d2 == m, col, N), axis=1, keepdims=True)
        cols.append(am)
        d2 = jnp.where(col == am, jnp.inf, d2)
    idx_ref[...] = jnp.concatenate(cols, axis=1) + b * N


def _knn_call(x, xt):
    return pl.pallas_call(
        _knn_body,
        grid=(B, N // BN),
        in_specs=[
            pl.BlockSpec((1, 3, N), lambda b, i: (b, 0, 0)),
            pl.BlockSpec((1, N, 3), lambda b, i: (b, 0, 0)),
        ],
        out_specs=pl.BlockSpec((BN, K), lambda b, i: (b * (N // BN) + i, 0)),
        out_shape=jax.ShapeDtypeStruct((M, K), jnp.int32),
    )(x, xt)


def _sc_gather_call(xrows, idxr):
    RW = xrows.shape[1]
    mesh = plsc.VectorSubcoreMesh(core_axis_name="c", subcore_axis_name="s",
                                  num_cores=NC, num_subcores=NS)

    def body(x_hbm, idx_hbm, nb_hbm, idxb0, idxb1, rows0, rows1,
             si0, si1, sg0, sg1):
        wid = lax.axis_index("s") * NC + lax.axis_index("c")
        base = wid * ROWS_W
        idxb = (idxb0, idxb1)
        rows = (rows0, rows1)
        si = (si0, si1)
        sg = (sg0, sg1)

        def issue_idx(s, j):
            pltpu.async_copy(
                idx_hbm.at[pl.ds(base + s * GCH, GCH)], idxb[j], si[j])

        def wait_idx(s, j):
            pltpu.make_async_copy(
                idx_hbm.at[pl.ds(base + s * GCH, GCH)], idxb[j], si[j]).wait()

        def issue(s, j):
            pltpu.async_copy(x_hbm.at[idxb[j]], rows[j], sg[j])

        def wait(s, j):
            pltpu.make_async_copy(x_hbm.at[idxb[j]], rows[j], sg[j]).wait()

        issue_idx(0, 0)
        wait_idx(0, 0)
        issue(0, 0)
        issue_idx(1, 1)

        def pair(i, carry):
            for j in range(2):
                s = 2 * i + j
                nxt = 1 - j

                wait(s, j)

                @pl.when(s + 1 < GS)
                def _():
                    wait_idx(s + 1, nxt)
                    issue(s + 1, nxt)

                @pl.when(s + 2 < GS)
                def _():
                    issue_idx(s + 2, j)

                pltpu.sync_copy(rows[j], nb_hbm.at[pl.ds(base + s * GCH, GCH)])
            return carry

        lax.fori_loop(0, GS // 2, pair, 0)

    fn = pl.kernel(
        body,
        out_type=jax.ShapeDtypeStruct((M * K, RW), jnp.float32),
        mesh=mesh,
        scratch_types=[
            pltpu.VMEM((GCH,), jnp.int32),
            pltpu.VMEM((GCH,), jnp.int32),
            pltpu.VMEM((GCH, RW), jnp.float32),
            pltpu.VMEM((GCH, RW), jnp.float32),
            pltpu.SemaphoreType.DMA,
            pltpu.SemaphoreType.DMA,
            pltpu.SemaphoreType.DMA,
            pltpu.SemaphoreType.DMA,
        ],
    )
    return fn(xrows, idxr)


def _make_conv_body(C, O, OW):
    def body(nb_ref, x_ref, w_ref, wd_ref, out_ref):
        xe = x_ref[...]
        hs = []
        for c in range(3):
            xc = xe[:, c * C:(c + 1) * C]
            nbc = nb_ref[:, c * C:(c + 1) * C]
            xcb = jnp.broadcast_to(xc[:, None, :], (RB, K, C))
            yd = (nbc.reshape(RB, K, C) - xcb).reshape(RB * K, C)
            ycat = jnp.concatenate([yd, xcb.reshape(RB * K, C)], axis=1)
            h = jnp.dot(ycat, w_ref[...],
                        preferred_element_type=jnp.float32)
            hs.append(h)
        ds = [jnp.dot(h, wd_ref[...], preferred_element_type=jnp.float32)
              for h in hs]
        hx, hy, hz = hs
        dx, dy, dz = ds
        den = jnp.sqrt(dx * dx + dy * dy + dz * dz) + EPS
        dnx = dx / den
        dny = dy / den
        dnz = dz / den
        dt = hx * dnx + hy * dny + hz * dnz
        acted = jnp.where(dt >= 0, dt, 0.2 * dt)
        t = acted - dt
        for c, (h, dn) in enumerate(((hx, dnx), (hy, dny), (hz, dnz))):
            o3 = (h + t * dn).reshape(RB, K, O)
            acc = o3[:, 0, :]
            for k in range(1, K):
                acc = acc + o3[:, k, :]
            out_ref[:, c * O:(c + 1) * O] = acc * (1.0 / K)
        if OW > 3 * O:
            out_ref[:, 3 * O:] = jnp.zeros((RB, OW - 3 * O), jnp.float32)
    return body


def _conv_call(nb, xrows, wt, wdt, O):
    C = wt.shape[0] // 2
    RW = xrows.shape[1]
    OW = _r128(3 * O)
    nblk = M // RB
    return pl.pallas_call(
        _make_conv_body(C, O, OW),
        grid=(nblk,),
        in_specs=[
            pl.BlockSpec((RB * K, RW), lambda i: (i, 0)),
            pl.BlockSpec((RB, RW), lambda i: (i, 0)),
            pl.BlockSpec((2 * C, O), lambda i: (0, 0)),
            pl.BlockSpec((O, O), lambda i: (0, 0)),
        ],
        out_specs=pl.BlockSpec((RB, OW), lambda i: (i, 0)),
        out_shape=jax.ShapeDtypeStruct((M, OW), jnp.float32),
    )(nb, xrows, wt, wdt)


CHC = (32, 64, 128, 128)


def _convc_body(x1_ref, x2_ref, x3_ref, x4_ref,
                w1_ref, w2_ref, w3_ref, w4_ref, wd_ref, out_ref):
    i = pl.program_id(1)
    xs = (x1_ref, x2_ref, x3_ref, x4_ref)
    wcat = jnp.concatenate([w1_ref[...], w2_ref[...], w3_ref[...],
                            w4_ref[...]], axis=0)
    hs = []
    ds = []
    for c in range(3):
        xcat = jnp.concatenate(
            [xr[:, c * cl:(c + 1) * cl] for xr, cl in zip(xs, CHC)], axis=1)
        hc = jnp.dot(xcat, wcat, preferred_element_type=jnp.float32)
        hs.append(hc)
        ds.append(jnp.dot(hc, wd_ref[...],
                          preferred_element_type=jnp.float32))
    hx, hy, hz = hs
    dx, dy, dz = ds
    den = jnp.sqrt(dx * dx + dy * dy + dz * dz) + EPS
    dnx = dx / den
    dny = dy / den
    dnz = dz / den
    dt = hx * dnx + hy * dny + hz * dnz
    acted = jnp.where(dt >= 0, dt, 0.2 * dt)
    t = acted - dt

    @pl.when(i == 0)
    def _():
        out_ref[...] = jnp.zeros_like(out_ref)

    out_ref[:, 0, :] += jnp.sum(hx + t * dnx, axis=0, keepdims=True)
    out_ref[:, 1, :] += jnp.sum(hy + t * dny, axis=0, keepdims=True)
    out_ref[:, 2, :] += jnp.sum(hz + t * dnz, axis=0, keepdims=True)


def _convc_call(x1, x2, x3, x4, w1, w2, w3, w4, wd):
    nb = N // RBC

    def rspec(w):
        return pl.BlockSpec((RBC, w), lambda b, i: (b * nb + i, 0))

    def wspec(c):
        return pl.BlockSpec((c, 128), lambda b, i: (0, 0))

    return pl.pallas_call(
        _convc_body,
        grid=(B, nb),
        in_specs=[rspec(x1.shape[1]), rspec(x2.shape[1]),
                  rspec(x3.shape[1]), rspec(x4.shape[1]),
                  wspec(32), wspec(64), wspec(128), wspec(128),
                  pl.BlockSpec((128, 1), lambda b, i: (0, 0))],
        out_specs=pl.BlockSpec((1, 3, 128), lambda b, i: (b, 0, 0)),
        out_shape=jax.ShapeDtypeStruct((B, 3, 128), jnp.float32),
    )(x1, x2, x3, x4, w1, w2, w3, w4, wd)


def _vl(w, x):
    return jnp.einsum('oi,bi...->bo...', w, x)


def _vact(wdir, x):
    d = _vl(wdir, x)
    dn = d / (jnp.linalg.norm(d, axis=2, keepdims=True) + EPS)
    dot = jnp.sum(x * dn, axis=2, keepdims=True)
    acted = jnp.where(dot >= 0, dot, 0.2 * dot)
    return x + (acted - dot) * dn


def _vlna(p, prefix, x):
    return _vact(p[prefix + '_dir'], _vl(p[prefix + '_lin'], x))


def _cevn(x):
    n = jnp.linalg.norm(x, axis=2, keepdims=True)
    d = x / (n + EPS)
    nrm = n / (jnp.linalg.norm(n, axis=1, keepdims=True) + EPS)
    return d * nrm


def kernel(x, label, params):
    p = params
    xt = jnp.transpose(x, (0, 2, 1))
    idxg = _knn_call(x, xt)
    idxr = idxg.reshape(M * K)

    X = jnp.pad(xt.reshape(M, 3), ((0, 0), (0, 125)))
    feats = []
    for name, C, O in (('conv1', 1, 32), ('conv2', 32, 64),
                       ('conv3', 64, 128), ('conv4', 128, 128)):
        wt = jnp.transpose(p[name + '_lin'])
        wdt = jnp.transpose(p[name + '_dir'])
        nb = _sc_gather_call(X, idxr)
        X = _conv_call(nb, X, wt, wdt, O)
        feats.append(X)

    x1, x2, x3, x4 = feats
    wc = p['convc_lin']
    ygs = _convc_call(
        x1, x2, x3, x4,
        jnp.transpose(wc[:, :32]), jnp.transpose(wc[:, 32:96]),
        jnp.transpose(wc[:, 96:224]), jnp.transpose(wc[:, 224:]),
        jnp.transpose(p['convc_dir']))
    yg = jnp.transpose(ygs, (0, 2, 1)) * (1.0 / N)

    so3f = _cevn(yg)
    z_so3 = _vl(p['fc_O'], so3f)
    r_pred = jnp.swapaxes(z_so3, 2, 1)
    u, _, vh = jnp.linalg.svd(r_pred, full_matrices=False)
    so3 = jnp.swapaxes(jnp.einsum('bij,bjk->bik', u, vh), 2, 1)
    scale = jnp.mean(jnp.linalg.norm(yg, axis=-1) + 1e-12, axis=1) * SCALE_FACTOR
    xg = yg[..., None]
    h = _vlna(p, 'ctr0', xg)
    h = _vlna(p, 'ctr1', h)
    center = (_vl(p['ctr_out'], h) + _vl(p['ctr_sc'], xg))[..., 0] * SCALE_FACTOR
    yn = _cevn(yg)
    m = _vlna(p, 'mean2', _vlna(p, 'mean1', yn))
    lv = _vlna(p, 'logvar2', _vlna(p, 'logvar1', yn))
    so3_m = _vl(p['fc_inv_mean'], m[..., None])[..., 0]
    inv_mean = jnp.sum(m * so3_m, axis=-1)
    so3_lv = _vl(p['fc_inv_logvar'], lv[..., None])[..., 0]
    inv_logvar = jnp.sum(lv * so3_lv, axis=-1)
    return inv_mean, inv_logvar, so3, scale, center
